# baseline (speedup 1.0000x reference)
import functools

import jax
import jax.numpy as jnp
from jax import lax
from jax.experimental import pallas as pl
from jax.experimental.pallas import tpu as pltpu
from jax.experimental.pallas import tpu_sc as plsc

N, E0, E1, S1, S2, D = 10000, 320000, 160000, 5000, 1000, 128

NC, NS = 2, 16
NW = NC * NS
CHUNK = 128

S1_PAD = 5120
S2_PAD = 1024


def _make_edge_pass(n_edges, s_pad, s_clamp, RG, RI):
    nchunks = n_edges // CHUNK
    assert nchunks * CHUNK == n_edges
    zrows = s_pad // NS
    assert zrows * NS == s_pad and zrows % 8 == 0

    nbase = nchunks // NW
    extra = nchunks % NW
    max_ncw = nbase + (1 if extra else 0)
    GD = RG - 1
    assert RI % RG == 0
    ngroups = (max_ncw + RI - 1) // RI
    main_g = max(0, (nbase - RI + 1) // RI)
    assert nbase >= RI

    mesh = plsc.VectorSubcoreMesh(core_axis_name="c", subcore_axis_name="s")

    @functools.partial(
        pl.kernel,
        mesh=mesh,
        compiler_params=pltpu.CompilerParams(needs_layout_passes=False),
        out_type=(
            jax.ShapeDtypeStruct((NC, s_pad, D), jnp.float32),
            jax.ShapeDtypeStruct((NW, s_pad), jnp.float32),
        ),
        scratch_types=(
            [pltpu.VMEM((CHUNK,), jnp.int32) for _ in range(2 * RI)]
            + [pltpu.VMEM((CHUNK, D), jnp.float32) for _ in range(RG)]
            + [
                pltpu.VMEM((s_pad,), jnp.float32),
                pltpu.VMEM_SHARED((s_pad, D), jnp.float32),
            ]
            + [pltpu.SemaphoreType.DMA for _ in range(RI + RG)]
        ),
    )
    def edge_pass(src_hbm, row_hbm, col_hbm, zsum_hbm, zcnt_hbm,
                  sum_out, cnt_out, *scratch):
        ridx = list(scratch[0:RI])
        cidx = list(scratch[RI:2 * RI])
        rows = list(scratch[2 * RI:2 * RI + RG])
        cnt = scratch[2 * RI + RG]
        acc = scratch[2 * RI + RG + 1]
        sem_i = list(scratch[2 * RI + RG + 2:2 * RI + RG + 2 + RI])
        sem_g = list(scratch[2 * RI + RG + 2 + RI:])

        c = lax.axis_index("c")
        s = lax.axis_index("s")
        wid = s * NC + c
        ones = jnp.full((16,), 1.0, jnp.float32)
        ncw = nbase + jnp.where(wid < extra, 1, 0)

        def idx_copies(j, b):
            base = (wid + j * NW) * CHUNK
            return (
                pltpu.make_async_copy(row_hbm.at[pl.ds(base, CHUNK)],
                                      ridx[b], sem_i[b]),
                pltpu.make_async_copy(col_hbm.at[pl.ds(base, CHUNK)],
                                      cidx[b], sem_i[b]),
            )

        def gather_copy(b):
            return pltpu.make_async_copy(src_hbm.at[ridx[b]],
                                         rows[b % RG], sem_g[b % RG])

        def stage_a(j, b):
            for d in idx_copies(j + GD, (b + GD) % RI):
                d.wait()
            gather_copy((b + GD) % RI).start()

        def stage_b(j, b):
            for i in range(CHUNK // 16):
                iv = jnp.minimum(cidx[b][pl.ds(i * 16, 16)], s_clamp - 1)
                cidx[b][pl.ds(i * 16, 16)] = iv
                plsc.addupdate_scatter(cnt, [iv], ones)
            gather_copy(b).wait()
            pltpu.sync_copy(rows[b % RG], acc.at[cidx[b]], add=True)

        def stage_c(j, b):
            for d in idx_copies(j + RI - 1, (b + RI - 1) % RI):
                d.start()

        for k in range(RI - 1):
            for d in idx_copies(k, k):
                d.start()
        pltpu.sync_copy(zcnt_hbm, cnt)
        pltpu.sync_copy(zsum_hbm, acc.at[pl.ds(s * zrows, zrows)])
        for k in range(GD):
            for d in idx_copies(k, k):
                d.wait()
            gather_copy(k).start()
        plsc.subcore_barrier()

        def group_main(g, carry):
            for b in range(RI):
                j = g * RI + b
                stage_a(j, b)
                stage_b(j, b)
                stage_c(j, b)
            return carry

        def group_tail(g, carry):
            for b in range(RI):
                j = g * RI + b
                pl.when(j + GD < ncw)(lambda: stage_a(j, b))
                pl.when(j < ncw)(lambda: stage_b(j, b))
                pl.when(j + RI - 1 < ncw)(lambda: stage_c(j, b))
            return carry

        lax.fori_loop(0, main_g, group_main, 0)
        lax.fori_loop(main_g, ngroups, group_tail, 0)
        pltpu.sync_copy(cnt, cnt_out.at[wid])
        plsc.subcore_barrier()
        pltpu.sync_copy(acc.at[pl.ds(s * zrows, zrows)],
                        sum_out.at[c, pl.ds(s * zrows, zrows)])

    return edge_pass


_edge_pass0 = _make_edge_pass(E0, S1_PAD, S1, 4, 8)
_edge_pass1 = _make_edge_pass(E1, S2_PAD, S2, 6, 12)


def _dense_body(last, p_ref, c_ref, xt_ref, wl_ref, bl_ref, wr_ref, o_ref):
    sums = p_ref[0] + p_ref[1]
    cnt = jnp.sum(c_ref[...], axis=0)[:, None]
    aggr = sums / jnp.maximum(cnt, 1.0)
    h = (jnp.dot(aggr, wl_ref[...], preferred_element_type=jnp.float32)
         + bl_ref[...]
         + jnp.dot(xt_ref[...], wr_ref[...], preferred_element_type=jnp.float32))
    if last:
        m = jnp.max(h, axis=-1, keepdims=True)
        o_ref[...] = (h - m) - jnp.log(
            jnp.sum(jnp.exp(h - m), axis=-1, keepdims=True))
    else:
        o_ref[...] = jnp.maximum(h, 0.0)


def _dense_layer(p, c, xt, wl, bl, wr, n_rows, last):
    blk = 1024
    grid = (n_rows + blk - 1) // blk
    return pl.pallas_call(
        functools.partial(_dense_body, last),
        grid=(grid,),
        in_specs=[
            pl.BlockSpec((NC, blk, D), lambda i: (0, i, 0)),
            pl.BlockSpec((NW, blk), lambda i: (0, i)),
            pl.BlockSpec((blk, D), lambda i: (i, 0)),
            pl.BlockSpec((D, D), lambda i: (0, 0)),
            pl.BlockSpec((1, D), lambda i: (0, 0)),
            pl.BlockSpec((D, D), lambda i: (0, 0)),
        ],
        out_specs=pl.BlockSpec((blk, D), lambda i: (i, 0)),
        out_shape=jax.ShapeDtypeStruct((n_rows, D), jnp.float32),
    )(p, c, xt, wl, bl, wr)


def kernel(x, row0, col0, row1, col1, size1, size2, Wl0, bl0, Wr0, Wl1, bl1, Wr1):
    zsum0 = jnp.zeros((S1_PAD // NS, D), jnp.float32)
    zcnt0 = jnp.zeros((S1_PAD,), jnp.float32)
    zsum1 = jnp.zeros((S2_PAD // NS, D), jnp.float32)
    zcnt1 = jnp.zeros((S2_PAD,), jnp.float32)

    p0, c0 = _edge_pass0(x, row0, col0, zsum0, zcnt0)
    h = _dense_layer(p0, c0, x, Wl0, bl0.reshape(1, D), Wr0, S1, last=False)
    p1, c1 = _edge_pass1(h, row1, col1, zsum1, zcnt1)
    out = _dense_layer(p1, c1, h, Wl1, bl1.reshape(1, D), Wr1, S2, last=True)
    return out


# speedup vs baseline: 1.0167x; 1.0167x over previous
import functools

import jax
import jax.numpy as jnp
from jax import lax
from jax.experimental import pallas as pl
from jax.experimental.pallas import tpu as pltpu
from jax.experimental.pallas import tpu_sc as plsc

N, E0, E1, S1, S2, D = 10000, 320000, 160000, 5000, 1000, 128

NC, NS = 2, 16
NW = NC * NS
CHUNK = 128

S1_PAD = 5120
S2_PAD = 1024


def _make_edge_pass(n_edges, s_pad, s_clamp, RG, RI):
    nchunks = n_edges // CHUNK
    assert nchunks * CHUNK == n_edges
    zrows = s_pad // NS
    assert zrows * NS == s_pad and zrows % 8 == 0

    nbase = nchunks // NW
    extra = nchunks % NW
    max_ncw = nbase + (1 if extra else 0)
    GD = RG - 1
    assert RI % RG == 0
    ngroups = (max_ncw + RI - 1) // RI
    main_g = max(0, (nbase - RI + 1) // RI)
    assert nbase >= RI

    mesh = plsc.VectorSubcoreMesh(core_axis_name="c", subcore_axis_name="s")

    @functools.partial(
        pl.kernel,
        mesh=mesh,
        compiler_params=pltpu.CompilerParams(needs_layout_passes=False),
        out_type=(
            jax.ShapeDtypeStruct((NC, s_pad, D), jnp.float32),
            jax.ShapeDtypeStruct((NW, s_pad), jnp.float32),
        ),
        scratch_types=(
            [pltpu.VMEM((CHUNK,), jnp.int32) for _ in range(2 * RI)]
            + [pltpu.VMEM((CHUNK, D), jnp.float32) for _ in range(RG)]
            + [
                pltpu.VMEM((s_pad,), jnp.float32),
                pltpu.VMEM_SHARED((s_pad, D), jnp.float32),
            ]
            + [pltpu.SemaphoreType.DMA for _ in range(RI + RG)]
        ),
    )
    def edge_pass(src_hbm, row_hbm, col_hbm, zsum_hbm, zcnt_hbm,
                  sum_out, cnt_out, *scratch):
        ridx = list(scratch[0:RI])
        cidx = list(scratch[RI:2 * RI])
        rows = list(scratch[2 * RI:2 * RI + RG])
        cnt = scratch[2 * RI + RG]
        acc = scratch[2 * RI + RG + 1]
        sem_i = list(scratch[2 * RI + RG + 2:2 * RI + RG + 2 + RI])
        sem_g = list(scratch[2 * RI + RG + 2 + RI:])

        c = lax.axis_index("c")
        s = lax.axis_index("s")
        wid = s * NC + c
        ones = jnp.full((16,), 1.0, jnp.float32)
        ncw = nbase + jnp.where(wid < extra, 1, 0)

        def idx_copies(j, b):
            base = (wid + j * NW) * CHUNK
            return (
                pltpu.make_async_copy(row_hbm.at[pl.ds(base, CHUNK)],
                                      ridx[b], sem_i[b]),
                pltpu.make_async_copy(col_hbm.at[pl.ds(base, CHUNK)],
                                      cidx[b], sem_i[b]),
            )

        def gather_copy(b):
            return pltpu.make_async_copy(src_hbm.at[ridx[b]],
                                         rows[b % RG], sem_g[b % RG])

        def stage_a(j, b):
            for d in idx_copies(j + GD, (b + GD) % RI):
                d.wait()
            gather_copy((b + GD) % RI).start()

        def stage_b(j, b):
            for i in range(CHUNK // 16):
                iv = jnp.minimum(cidx[b][pl.ds(i * 16, 16)], s_clamp - 1)
                cidx[b][pl.ds(i * 16, 16)] = iv
                plsc.addupdate_scatter(cnt, [iv], ones)
            gather_copy(b).wait()
            pltpu.sync_copy(rows[b % RG], acc.at[cidx[b]], add=True)

        def stage_c(j, b):
            for d in idx_copies(j + RI - 1, (b + RI - 1) % RI):
                d.start()

        for k in range(RI - 1):
            for d in idx_copies(k, k):
                d.start()
        pltpu.sync_copy(zcnt_hbm, cnt)
        pltpu.sync_copy(zsum_hbm, acc.at[pl.ds(s * zrows, zrows)])
        for k in range(GD):
            for d in idx_copies(k, k):
                d.wait()
            gather_copy(k).start()
        plsc.subcore_barrier()

        def group_main(g, carry):
            for b in range(RI):
                j = g * RI + b
                stage_a(j, b)
                stage_b(j, b)
                stage_c(j, b)
            return carry

        def group_tail(g, carry):
            for b in range(RI):
                j = g * RI + b
                pl.when(j + GD < ncw)(lambda: stage_a(j, b))
                pl.when(j < ncw)(lambda: stage_b(j, b))
                pl.when(j + RI - 1 < ncw)(lambda: stage_c(j, b))
            return carry

        lax.fori_loop(0, main_g, group_main, 0)
        lax.fori_loop(main_g, ngroups, group_tail, 0)
        pltpu.sync_copy(cnt, cnt_out.at[wid])
        plsc.subcore_barrier()
        pltpu.sync_copy(acc.at[pl.ds(s * zrows, zrows)],
                        sum_out.at[c, pl.ds(s * zrows, zrows)])

    return edge_pass


_edge_pass0 = _make_edge_pass(E0, S1_PAD, S1, 4, 8)
_edge_pass1 = _make_edge_pass(E1, S2_PAD, S2, 4, 8)


def _dense_body(last, p_ref, c_ref, xt_ref, wl_ref, bl_ref, wr_ref, o_ref):
    sums = p_ref[0] + p_ref[1]
    cnt = jnp.sum(c_ref[...], axis=0)[:, None]
    aggr = sums / jnp.maximum(cnt, 1.0)
    h = (jnp.dot(aggr, wl_ref[...], preferred_element_type=jnp.float32)
         + bl_ref[...]
         + jnp.dot(xt_ref[...], wr_ref[...], preferred_element_type=jnp.float32))
    if last:
        m = jnp.max(h, axis=-1, keepdims=True)
        o_ref[...] = (h - m) - jnp.log(
            jnp.sum(jnp.exp(h - m), axis=-1, keepdims=True))
    else:
        o_ref[...] = jnp.maximum(h, 0.0)


def _dense_layer(p, c, xt, wl, bl, wr, n_rows, last):
    blk = 1024
    grid = (n_rows + blk - 1) // blk
    return pl.pallas_call(
        functools.partial(_dense_body, last),
        grid=(grid,),
        in_specs=[
            pl.BlockSpec((NC, blk, D), lambda i: (0, i, 0)),
            pl.BlockSpec((NW, blk), lambda i: (0, i)),
            pl.BlockSpec((blk, D), lambda i: (i, 0)),
            pl.BlockSpec((D, D), lambda i: (0, 0)),
            pl.BlockSpec((1, D), lambda i: (0, 0)),
            pl.BlockSpec((D, D), lambda i: (0, 0)),
        ],
        out_specs=pl.BlockSpec((blk, D), lambda i: (i, 0)),
        out_shape=jax.ShapeDtypeStruct((n_rows, D), jnp.float32),
    )(p, c, xt, wl, bl, wr)


def kernel(x, row0, col0, row1, col1, size1, size2, Wl0, bl0, Wr0, Wl1, bl1, Wr1):
    zsum0 = jnp.zeros((S1_PAD // NS, D), jnp.float32)
    zcnt0 = jnp.zeros((S1_PAD,), jnp.float32)
    zsum1 = jnp.zeros((S2_PAD // NS, D), jnp.float32)
    zcnt1 = jnp.zeros((S2_PAD,), jnp.float32)

    p0, c0 = _edge_pass0(x, row0, col0, zsum0, zcnt0)
    h = _dense_layer(p0, c0, x, Wl0, bl0.reshape(1, D), Wr0, S1, last=False)
    p1, c1 = _edge_pass1(h, row1, col1, zsum1, zcnt1)
    out = _dense_layer(p1, c1, h, Wl1, bl1.reshape(1, D), Wr1, S2, last=True)
    return out


# speedup vs baseline: 1.0169x; 1.0002x over previous
import functools

import jax
import jax.numpy as jnp
from jax import lax
from jax.experimental import pallas as pl
from jax.experimental.pallas import tpu as pltpu
from jax.experimental.pallas import tpu_sc as plsc

N, E0, E1, S1, S2, D = 10000, 320000, 160000, 5000, 1000, 128

NC, NS = 2, 16
NW = NC * NS
CHUNK = 128

S1_PAD = 5120
S2_PAD = 1024


def _make_edge_pass(n_edges, s_pad, s_clamp, RG, RI):
    nchunks = n_edges // CHUNK
    assert nchunks * CHUNK == n_edges
    zrows = s_pad // NS
    assert zrows * NS == s_pad and zrows % 8 == 0

    nbase = nchunks // NW
    extra = nchunks % NW
    max_ncw = nbase + (1 if extra else 0)
    GD = RG - 1
    assert RI % RG == 0
    ngroups = (max_ncw + RI - 1) // RI
    main_g = max(0, (nbase - RI + 1) // RI)
    assert nbase >= RI

    mesh = plsc.VectorSubcoreMesh(core_axis_name="c", subcore_axis_name="s")

    @functools.partial(
        pl.kernel,
        mesh=mesh,
        compiler_params=pltpu.CompilerParams(needs_layout_passes=False),
        out_type=(
            jax.ShapeDtypeStruct((NC, s_pad, D), jnp.float32),
            jax.ShapeDtypeStruct((NW, s_pad), jnp.float32),
        ),
        scratch_types=(
            [pltpu.VMEM((CHUNK,), jnp.int32) for _ in range(2 * RI)]
            + [pltpu.VMEM((CHUNK, D), jnp.float32) for _ in range(RG)]
            + [
                pltpu.VMEM((s_pad,), jnp.float32),
                pltpu.VMEM_SHARED((s_pad, D), jnp.float32),
            ]
            + [pltpu.SemaphoreType.DMA for _ in range(RI + RG + 1)]
        ),
    )
    def edge_pass(src_hbm, row_hbm, col_hbm, zsum_hbm, zcnt_hbm,
                  sum_out, cnt_out, *scratch):
        ridx = list(scratch[0:RI])
        cidx = list(scratch[RI:2 * RI])
        rows = list(scratch[2 * RI:2 * RI + RG])
        cnt = scratch[2 * RI + RG]
        acc = scratch[2 * RI + RG + 1]
        sem_i = list(scratch[2 * RI + RG + 2:2 * RI + RG + 2 + RI])
        sem_g = list(scratch[2 * RI + RG + 2 + RI:2 * RI + 2 * RG + 2 + RI])
        sem_s = scratch[2 * RI + 2 * RG + 2 + RI]

        c = lax.axis_index("c")
        s = lax.axis_index("s")
        wid = s * NC + c
        ones = jnp.full((16,), 1.0, jnp.float32)
        ncw = nbase + jnp.where(wid < extra, 1, 0)

        def idx_copies(j, b):
            base = (wid + j * NW) * CHUNK
            return (
                pltpu.make_async_copy(row_hbm.at[pl.ds(base, CHUNK)],
                                      ridx[b], sem_i[b]),
                pltpu.make_async_copy(col_hbm.at[pl.ds(base, CHUNK)],
                                      cidx[b], sem_i[b]),
            )

        def gather_copy(b):
            return pltpu.make_async_copy(src_hbm.at[ridx[b]],
                                         rows[b % RG], sem_g[b % RG])

        def stage_a(j, b):
            for d in idx_copies(j + GD, (b + GD) % RI):
                d.wait()
            gather_copy((b + GD) % RI).start()

        def stage_b(j, b):
            for i in range(CHUNK // 16):
                iv = jnp.minimum(cidx[b][pl.ds(i * 16, 16)], s_clamp - 1)
                cidx[b][pl.ds(i * 16, 16)] = iv
                plsc.addupdate_scatter(cnt, [iv], ones)
            gather_copy(b).wait()
            pltpu.async_copy(rows[b % RG], acc.at[cidx[b]], sem_s, add=True)

        def wait_scatter(b):
            pltpu.make_async_copy(rows[b % RG], acc.at[cidx[b]], sem_s).wait()

        def stage_c(j, b):
            for d in idx_copies(j + RI - 1, (b + RI - 1) % RI):
                d.start()

        for k in range(RI - 1):
            for d in idx_copies(k, k):
                d.start()
        pltpu.sync_copy(zcnt_hbm, cnt)
        pltpu.sync_copy(zsum_hbm, acc.at[pl.ds(s * zrows, zrows)])
        for k in range(GD):
            for d in idx_copies(k, k):
                d.wait()
            gather_copy(k).start()
        plsc.subcore_barrier()

        def group_main(g, carry):
            for b in range(RI):
                j = g * RI + b
                if b == 0:
                    pl.when(g > 0)(lambda: wait_scatter(RI - 1))
                else:
                    wait_scatter(b - 1)
                stage_a(j, b)
                stage_b(j, b)
                stage_c(j, b)
            return carry

        def group_tail(g, carry):
            for b in range(RI):
                j = g * RI + b
                pl.when((j >= 1) & (j < ncw))(
                    lambda: wait_scatter((b - 1) % RI))
                pl.when(j + GD < ncw)(lambda: stage_a(j, b))
                pl.when(j < ncw)(lambda: stage_b(j, b))
                pl.when(j + RI - 1 < ncw)(lambda: stage_c(j, b))
            return carry

        lax.fori_loop(0, main_g, group_main, 0)
        lax.fori_loop(main_g, ngroups, group_tail, 0)
        wait_scatter(0)
        pltpu.sync_copy(cnt, cnt_out.at[wid])
        plsc.subcore_barrier()
        pltpu.sync_copy(acc.at[pl.ds(s * zrows, zrows)],
                        sum_out.at[c, pl.ds(s * zrows, zrows)])

    return edge_pass


_edge_pass0 = _make_edge_pass(E0, S1_PAD, S1, 4, 8)
_edge_pass1 = _make_edge_pass(E1, S2_PAD, S2, 4, 8)


def _dense_body(last, p_ref, c_ref, xt_ref, wl_ref, bl_ref, wr_ref, o_ref):
    sums = p_ref[0] + p_ref[1]
    cnt = jnp.sum(c_ref[...], axis=0)[:, None]
    aggr = sums / jnp.maximum(cnt, 1.0)
    h = (jnp.dot(aggr, wl_ref[...], preferred_element_type=jnp.float32)
         + bl_ref[...]
         + jnp.dot(xt_ref[...], wr_ref[...], preferred_element_type=jnp.float32))
    if last:
        m = jnp.max(h, axis=-1, keepdims=True)
        o_ref[...] = (h - m) - jnp.log(
            jnp.sum(jnp.exp(h - m), axis=-1, keepdims=True))
    else:
        o_ref[...] = jnp.maximum(h, 0.0)


def _dense_layer(p, c, xt, wl, bl, wr, n_rows, last):
    blk = 1024
    grid = (n_rows + blk - 1) // blk
    return pl.pallas_call(
        functools.partial(_dense_body, last),
        grid=(grid,),
        in_specs=[
            pl.BlockSpec((NC, blk, D), lambda i: (0, i, 0)),
            pl.BlockSpec((NW, blk), lambda i: (0, i)),
            pl.BlockSpec((blk, D), lambda i: (i, 0)),
            pl.BlockSpec((D, D), lambda i: (0, 0)),
            pl.BlockSpec((1, D), lambda i: (0, 0)),
            pl.BlockSpec((D, D), lambda i: (0, 0)),
        ],
        out_specs=pl.BlockSpec((blk, D), lambda i: (i, 0)),
        out_shape=jax.ShapeDtypeStruct((n_rows, D), jnp.float32),
    )(p, c, xt, wl, bl, wr)


def kernel(x, row0, col0, row1, col1, size1, size2, Wl0, bl0, Wr0, Wl1, bl1, Wr1):
    zsum0 = jnp.zeros((S1_PAD // NS, D), jnp.float32)
    zcnt0 = jnp.zeros((S1_PAD,), jnp.float32)
    zsum1 = jnp.zeros((S2_PAD // NS, D), jnp.float32)
    zcnt1 = jnp.zeros((S2_PAD,), jnp.float32)

    p0, c0 = _edge_pass0(x, row0, col0, zsum0, zcnt0)
    h = _dense_layer(p0, c0, x, Wl0, bl0.reshape(1, D), Wr0, S1, last=False)
    p1, c1 = _edge_pass1(h, row1, col1, zsum1, zcnt1)
    out = _dense_layer(p1, c1, h, Wl1, bl1.reshape(1, D), Wr1, S2, last=True)
    return out


# speedup vs baseline: 1.0169x; 1.0000x over previous
import functools

import jax
import jax.numpy as jnp
from jax import lax
from jax.experimental import pallas as pl
from jax.experimental.pallas import tpu as pltpu
from jax.experimental.pallas import tpu_sc as plsc

N, E0, E1, S1, S2, D = 10000, 320000, 160000, 5000, 1000, 128

NC, NS = 2, 16
NW = NC * NS
CHUNK = 128

S1_PAD = 5120
S2_PAD = 1024


def _make_edge_pass(n_edges, s_pad, s_clamp, RG, RI):
    nchunks = n_edges // CHUNK
    assert nchunks * CHUNK == n_edges
    zrows = s_pad // NS
    assert zrows * NS == s_pad and zrows % 8 == 0

    nbase = nchunks // NW
    extra = nchunks % NW
    max_ncw = nbase + (1 if extra else 0)
    GD = RG - 1
    assert RI % RG == 0
    ngroups = (max_ncw + RI - 1) // RI
    main_g = max(0, (nbase - RI + 1) // RI)
    assert nbase >= RI

    mesh = plsc.VectorSubcoreMesh(core_axis_name="c", subcore_axis_name="s")

    @functools.partial(
        pl.kernel,
        mesh=mesh,
        compiler_params=pltpu.CompilerParams(needs_layout_passes=False),
        out_type=(
            jax.ShapeDtypeStruct((NC, s_pad, D), jnp.float32),
            jax.ShapeDtypeStruct((NW, s_pad), jnp.float32),
        ),
        scratch_types=(
            [pltpu.VMEM((CHUNK,), jnp.int32) for _ in range(2 * RI)]
            + [pltpu.VMEM((CHUNK, D), jnp.float32) for _ in range(RG)]
            + [
                pltpu.VMEM((s_pad,), jnp.float32),
                pltpu.VMEM_SHARED((s_pad, D), jnp.float32),
            ]
            + [pltpu.SemaphoreType.DMA for _ in range(RI + RG)]
        ),
    )
    def edge_pass(src_hbm, row_hbm, col_hbm, zsum_hbm, zcnt_hbm,
                  sum_out, cnt_out, *scratch):
        ridx = list(scratch[0:RI])
        cidx = list(scratch[RI:2 * RI])
        rows = list(scratch[2 * RI:2 * RI + RG])
        cnt = scratch[2 * RI + RG]
        acc = scratch[2 * RI + RG + 1]
        sem_i = list(scratch[2 * RI + RG + 2:2 * RI + RG + 2 + RI])
        sem_g = list(scratch[2 * RI + RG + 2 + RI:])

        c = lax.axis_index("c")
        s = lax.axis_index("s")
        wid = s * NC + c
        ones = jnp.full((16,), 1.0, jnp.float32)
        ncw = nbase + jnp.where(wid < extra, 1, 0)

        def idx_copies(j, b):
            base = (wid + j * NW) * CHUNK
            return (
                pltpu.make_async_copy(row_hbm.at[pl.ds(base, CHUNK)],
                                      ridx[b], sem_i[b]),
                pltpu.make_async_copy(col_hbm.at[pl.ds(base, CHUNK)],
                                      cidx[b], sem_i[b]),
            )

        def gather_copy(b):
            return pltpu.make_async_copy(src_hbm.at[ridx[b]],
                                         rows[b % RG], sem_g[b % RG])

        def stage_a(j, b):
            for d in idx_copies(j + GD, (b + GD) % RI):
                d.wait()
            gather_copy((b + GD) % RI).start()

        def stage_b(j, b):
            for i in range(CHUNK // 16):
                iv = jnp.minimum(cidx[b][pl.ds(i * 16, 16)], s_clamp - 1)
                cidx[b][pl.ds(i * 16, 16)] = iv
                plsc.addupdate_scatter(cnt, [iv], ones)
            gather_copy(b).wait()
            pltpu.sync_copy(rows[b % RG], acc.at[cidx[b]], add=True)

        def stage_c(j, b):
            for d in idx_copies(j + RI - 1, (b + RI - 1) % RI):
                d.start()

        for k in range(RI - 1):
            for d in idx_copies(k, k):
                d.start()
        pltpu.sync_copy(zcnt_hbm, cnt)
        pltpu.sync_copy(zsum_hbm, acc.at[pl.ds(s * zrows, zrows)])
        for k in range(GD):
            for d in idx_copies(k, k):
                d.wait()
            gather_copy(k).start()
        plsc.subcore_barrier()

        def group_main(g, carry):
            for b in range(RI):
                j = g * RI + b
                stage_a(j, b)
                stage_b(j, b)
                stage_c(j, b)
            return carry

        def group_tail(g, carry):
            for b in range(RI):
                j = g * RI + b
                pl.when(j + GD < ncw)(lambda: stage_a(j, b))
                pl.when(j < ncw)(lambda: stage_b(j, b))
                pl.when(j + RI - 1 < ncw)(lambda: stage_c(j, b))
            return carry

        lax.fori_loop(0, main_g, group_main, 0)
        lax.fori_loop(main_g, ngroups, group_tail, 0)
        pltpu.sync_copy(cnt, cnt_out.at[wid])
        plsc.subcore_barrier()
        pltpu.sync_copy(acc.at[pl.ds(s * zrows, zrows)],
                        sum_out.at[c, pl.ds(s * zrows, zrows)])

    return edge_pass


_edge_pass0 = _make_edge_pass(E0, S1_PAD, S1, 4, 8)
_edge_pass1 = _make_edge_pass(E1, S2_PAD, S2, 4, 8)


def _dense_body(last, p_ref, c_ref, xt_ref, wl_ref, bl_ref, wr_ref, o_ref):
    sums = p_ref[0] + p_ref[1]
    cnt = jnp.sum(c_ref[...], axis=0)[:, None]
    aggr = sums / jnp.maximum(cnt, 1.0)
    h = (jnp.dot(aggr, wl_ref[...], preferred_element_type=jnp.float32)
         + bl_ref[...]
         + jnp.dot(xt_ref[...], wr_ref[...], preferred_element_type=jnp.float32))
    if last:
        m = jnp.max(h, axis=-1, keepdims=True)
        o_ref[...] = (h - m) - jnp.log(
            jnp.sum(jnp.exp(h - m), axis=-1, keepdims=True))
    else:
        o_ref[...] = jnp.maximum(h, 0.0)


def _dense_layer(p, c, xt, wl, bl, wr, n_rows, last):
    blk = 1024
    grid = (n_rows + blk - 1) // blk
    return pl.pallas_call(
        functools.partial(_dense_body, last),
        grid=(grid,),
        in_specs=[
            pl.BlockSpec((NC, blk, D), lambda i: (0, i, 0)),
            pl.BlockSpec((NW, blk), lambda i: (0, i)),
            pl.BlockSpec((blk, D), lambda i: (i, 0)),
            pl.BlockSpec((D, D), lambda i: (0, 0)),
            pl.BlockSpec((1, D), lambda i: (0, 0)),
            pl.BlockSpec((D, D), lambda i: (0, 0)),
        ],
        out_specs=pl.BlockSpec((blk, D), lambda i: (i, 0)),
        out_shape=jax.ShapeDtypeStruct((n_rows, D), jnp.float32),
    )(p, c, xt, wl, bl, wr)


def kernel(x, row0, col0, row1, col1, size1, size2, Wl0, bl0, Wr0, Wl1, bl1, Wr1):
    zsum0 = jnp.zeros((S1_PAD // NS, D), jnp.float32)
    zcnt0 = jnp.zeros((S1_PAD,), jnp.float32)
    zsum1 = jnp.zeros((S2_PAD // NS, D), jnp.float32)
    zcnt1 = jnp.zeros((S2_PAD,), jnp.float32)

    p0, c0 = _edge_pass0(x, row0, col0, zsum0, zcnt0)
    h = _dense_layer(p0, c0, x, Wl0, bl0.reshape(1, D), Wr0, S1, last=False)
    p1, c1 = _edge_pass1(h, row1, col1, zsum1, zcnt1)
    out = _dense_layer(p1, c1, h, Wl1, bl1.reshape(1, D), Wr1, S2, last=True)
    return out


# speedup vs baseline: 1.0622x; 1.0446x over previous
import functools

import jax
import jax.numpy as jnp
from jax import lax
from jax.experimental import pallas as pl
from jax.experimental.pallas import tpu as pltpu
from jax.experimental.pallas import tpu_sc as plsc

N, E0, E1, S1, S2, D = 10000, 320000, 160000, 5000, 1000, 128

NC, NS = 2, 16
NW = NC * NS
CHUNK = 128

S1_PAD = 5120
S2_PAD = 1024


def _make_edge_pass(n_edges, s_pad, s_clamp, RG, RI):
    nchunks = n_edges // CHUNK
    assert nchunks * CHUNK == n_edges
    zrows = s_pad // NS
    assert zrows * NS == s_pad and zrows % 8 == 0

    nbase = nchunks // NW
    extra = nchunks % NW
    max_ncw = nbase + (1 if extra else 0)
    GD = RG - 1
    assert RI % RG == 0
    ngroups = (max_ncw + RI - 1) // RI
    main_g = max(0, (nbase - RI + 1) // RI)
    assert nbase >= RI

    mesh = plsc.VectorSubcoreMesh(core_axis_name="c", subcore_axis_name="s")

    @functools.partial(
        pl.kernel,
        mesh=mesh,
        compiler_params=pltpu.CompilerParams(needs_layout_passes=False),
        out_type=(
            jax.ShapeDtypeStruct((NC, s_pad, D), jnp.float32),
            jax.ShapeDtypeStruct((NW, s_pad), jnp.float32),
        ),
        scratch_types=(
            [pltpu.VMEM((CHUNK,), jnp.int32) for _ in range(2 * RI)]
            + [pltpu.VMEM((CHUNK, D), jnp.float32) for _ in range(RG)]
            + [
                pltpu.VMEM((s_pad,), jnp.float32),
                pltpu.VMEM_SHARED((s_pad, D), jnp.float32),
            ]
            + [pltpu.SemaphoreType.DMA for _ in range(RI + RG)]
        ),
    )
    def edge_pass(src_hbm, row_hbm, col_hbm, sum_out, cnt_out, *scratch):
        ridx = list(scratch[0:RI])
        cidx = list(scratch[RI:2 * RI])
        rows = list(scratch[2 * RI:2 * RI + RG])
        cnt = scratch[2 * RI + RG]
        acc = scratch[2 * RI + RG + 1]
        sem_i = list(scratch[2 * RI + RG + 2:2 * RI + RG + 2 + RI])
        sem_g = list(scratch[2 * RI + RG + 2 + RI:])

        c = lax.axis_index("c")
        s = lax.axis_index("s")
        wid = s * NC + c
        ones = jnp.full((16,), 1.0, jnp.float32)
        ncw = nbase + jnp.where(wid < extra, 1, 0)

        def idx_copies(j, b):
            base = (wid + j * NW) * CHUNK
            return (
                pltpu.make_async_copy(row_hbm.at[pl.ds(base, CHUNK)],
                                      ridx[b], sem_i[b]),
                pltpu.make_async_copy(col_hbm.at[pl.ds(base, CHUNK)],
                                      cidx[b], sem_i[b]),
            )

        def gather_copy(b):
            return pltpu.make_async_copy(src_hbm.at[ridx[b]],
                                         rows[b % RG], sem_g[b % RG])

        def stage_a(j, b):
            for d in idx_copies(j + GD, (b + GD) % RI):
                d.wait()
            gather_copy((b + GD) % RI).start()

        def stage_b(j, b):
            for i in range(CHUNK // 16):
                iv = jnp.minimum(cidx[b][pl.ds(i * 16, 16)], s_clamp - 1)
                cidx[b][pl.ds(i * 16, 16)] = iv
                plsc.addupdate_scatter(cnt, [iv], ones)
            gather_copy(b).wait()
            pltpu.sync_copy(rows[b % RG], acc.at[cidx[b]], add=True)

        def stage_c(j, b):
            for d in idx_copies(j + RI - 1, (b + RI - 1) % RI):
                d.start()

        for k in range(RI - 1):
            for d in idx_copies(k, k):
                d.start()
        zv = jnp.zeros((16,), jnp.float32)

        def zloop(i, carry):
            cnt[pl.ds(i * 16, 16)] = zv
            return carry

        lax.fori_loop(0, s_pad // 16, zloop, 0)

        def zrow(r, carry):
            for i in range(D // 16):
                rows[0][r, pl.ds(i * 16, 16)] = zv
            return carry

        lax.fori_loop(0, CHUNK, zrow, 0)
        done = 0
        while done < zrows:
            step = min(CHUNK, zrows - done)
            pltpu.sync_copy(rows[0].at[pl.ds(0, step)],
                            acc.at[pl.ds(s * zrows + done, step)])
            done += step
        for k in range(GD):
            for d in idx_copies(k, k):
                d.wait()
            gather_copy(k).start()
        plsc.subcore_barrier()

        def group_main(g, carry):
            for b in range(RI):
                j = g * RI + b
                stage_a(j, b)
                stage_b(j, b)
                stage_c(j, b)
            return carry

        def group_tail(g, carry):
            for b in range(RI):
                j = g * RI + b
                pl.when(j + GD < ncw)(lambda: stage_a(j, b))
                pl.when(j < ncw)(lambda: stage_b(j, b))
                pl.when(j + RI - 1 < ncw)(lambda: stage_c(j, b))
            return carry

        lax.fori_loop(0, main_g, group_main, 0)
        lax.fori_loop(main_g, ngroups, group_tail, 0)
        pltpu.sync_copy(cnt, cnt_out.at[wid])
        plsc.subcore_barrier()
        pltpu.sync_copy(acc.at[pl.ds(s * zrows, zrows)],
                        sum_out.at[c, pl.ds(s * zrows, zrows)])

    return edge_pass


_edge_pass0 = _make_edge_pass(E0, S1_PAD, S1, 4, 8)
_edge_pass1 = _make_edge_pass(E1, S2_PAD, S2, 4, 8)


def _dense_body(last, p_ref, c_ref, xt_ref, wl_ref, bl_ref, wr_ref, o_ref):
    sums = p_ref[0] + p_ref[1]
    cnt = jnp.sum(c_ref[...], axis=0)[:, None]
    aggr = sums / jnp.maximum(cnt, 1.0)
    h = (jnp.dot(aggr, wl_ref[...], preferred_element_type=jnp.float32)
         + bl_ref[...]
         + jnp.dot(xt_ref[...], wr_ref[...], preferred_element_type=jnp.float32))
    if last:
        m = jnp.max(h, axis=-1, keepdims=True)
        o_ref[...] = (h - m) - jnp.log(
            jnp.sum(jnp.exp(h - m), axis=-1, keepdims=True))
    else:
        o_ref[...] = jnp.maximum(h, 0.0)


def _dense_layer(p, c, xt, wl, bl, wr, n_rows, last):
    blk = 1024
    grid = (n_rows + blk - 1) // blk
    return pl.pallas_call(
        functools.partial(_dense_body, last),
        grid=(grid,),
        in_specs=[
            pl.BlockSpec((NC, blk, D), lambda i: (0, i, 0)),
            pl.BlockSpec((NW, blk), lambda i: (0, i)),
            pl.BlockSpec((blk, D), lambda i: (i, 0)),
            pl.BlockSpec((D, D), lambda i: (0, 0)),
            pl.BlockSpec((1, D), lambda i: (0, 0)),
            pl.BlockSpec((D, D), lambda i: (0, 0)),
        ],
        out_specs=pl.BlockSpec((blk, D), lambda i: (i, 0)),
        out_shape=jax.ShapeDtypeStruct((n_rows, D), jnp.float32),
    )(p, c, xt, wl, bl, wr)


def kernel(x, row0, col0, row1, col1, size1, size2, Wl0, bl0, Wr0, Wl1, bl1, Wr1):
    p0, c0 = _edge_pass0(x, row0, col0)
    h = _dense_layer(p0, c0, x, Wl0, bl0.reshape(1, D), Wr0, S1, last=False)
    p1, c1 = _edge_pass1(h, row1, col1)
    out = _dense_layer(p1, c1, h, Wl1, bl1.reshape(1, D), Wr1, S2, last=True)
    return out


# speedup vs baseline: 1.0848x; 1.0212x over previous
import functools

import jax
import jax.numpy as jnp
from jax import lax
from jax.experimental import pallas as pl
from jax.experimental.pallas import tpu as pltpu
from jax.experimental.pallas import tpu_sc as plsc

N, E0, E1, S1, S2, D = 10000, 320000, 160000, 5000, 1000, 128

NC, NS = 2, 16
NW = NC * NS
CHUNK = 128

S1_PAD = 5120
S2_PAD = 1024


def _make_edge_pass(n_edges, s_pad, s_clamp, RG, RI):
    nchunks = n_edges // CHUNK
    assert nchunks * CHUNK == n_edges
    zrows = s_pad // NS
    assert zrows * NS == s_pad and zrows % 8 == 0

    nbase = nchunks // NW
    extra = nchunks % NW
    max_ncw = nbase + (1 if extra else 0)
    GD = RG - 1
    assert RI % RG == 0
    ngroups = (max_ncw + RI - 1) // RI
    main_g = max(0, (nbase - RI + 1) // RI)
    assert nbase >= RI

    mesh = plsc.VectorSubcoreMesh(core_axis_name="c", subcore_axis_name="s")

    @functools.partial(
        pl.kernel,
        mesh=mesh,
        compiler_params=pltpu.CompilerParams(needs_layout_passes=False),
        out_type=(
            jax.ShapeDtypeStruct((NC, s_pad, D), jnp.float32),
            jax.ShapeDtypeStruct((NW, s_pad), jnp.float32),
        ),
        scratch_types=(
            [pltpu.VMEM((CHUNK,), jnp.int32) for _ in range(2 * RI)]
            + [pltpu.VMEM((CHUNK, D), jnp.float32) for _ in range(RG)]
            + [
                pltpu.VMEM((s_pad,), jnp.float32),
                pltpu.VMEM_SHARED((s_pad, D), jnp.float32),
            ]
            + [pltpu.SemaphoreType.DMA for _ in range(RI + RG)]
        ),
    )
    def edge_pass(src_hbm, row_hbm, col_hbm, sum_out, cnt_out, *scratch):
        ridx = list(scratch[0:RI])
        cidx = list(scratch[RI:2 * RI])
        rows = list(scratch[2 * RI:2 * RI + RG])
        cnt = scratch[2 * RI + RG]
        acc = scratch[2 * RI + RG + 1]
        sem_i = list(scratch[2 * RI + RG + 2:2 * RI + RG + 2 + RI])
        sem_g = list(scratch[2 * RI + RG + 2 + RI:])

        c = lax.axis_index("c")
        s = lax.axis_index("s")
        wid = s * NC + c
        ones = jnp.full((16,), 1.0, jnp.float32)
        ncw = nbase + jnp.where(wid < extra, 1, 0)

        def idx_copies(j, b):
            base = (wid + j * NW) * CHUNK
            return (
                pltpu.make_async_copy(row_hbm.at[pl.ds(base, CHUNK)],
                                      ridx[b], sem_i[b]),
                pltpu.make_async_copy(col_hbm.at[pl.ds(base, CHUNK)],
                                      cidx[b], sem_i[b]),
            )

        def gather_copy(b):
            return pltpu.make_async_copy(src_hbm.at[ridx[b]],
                                         rows[b % RG], sem_g[b % RG])

        def stage_a(j, b):
            for d in idx_copies(j + GD, (b + GD) % RI):
                d.wait()
            gather_copy((b + GD) % RI).start()

        def stage_b(j, b):
            for i in range(CHUNK // 16):
                iv = jnp.minimum(cidx[b][pl.ds(i * 16, 16)], s_clamp - 1)
                cidx[b][pl.ds(i * 16, 16)] = iv
                plsc.addupdate_scatter(cnt, [iv], ones)
            gather_copy(b).wait()
            pltpu.sync_copy(rows[b % RG], acc.at[cidx[b]], add=True)

        def stage_c(j, b):
            for d in idx_copies(j + RI - 1, (b + RI - 1) % RI):
                d.start()

        for k in range(RI - 1):
            for d in idx_copies(k, k):
                d.start()
        for k in range(GD):
            for d in idx_copies(k, k):
                d.wait()
            gather_copy(k).start()
        zv = jnp.zeros((16,), jnp.float32)
        zbuf = rows[RG - 1]

        def zloop(i, carry):
            cnt[pl.ds(i * 16, 16)] = zv
            return carry

        lax.fori_loop(0, s_pad // 16, zloop, 0)

        def zrow(r, carry):
            for i in range(D // 16):
                zbuf[r, pl.ds(i * 16, 16)] = zv
            return carry

        lax.fori_loop(0, min(CHUNK, zrows), zrow, 0)
        done = 0
        while done < zrows:
            step = min(CHUNK, zrows - done)
            pltpu.sync_copy(zbuf.at[pl.ds(0, step)],
                            acc.at[pl.ds(s * zrows + done, step)])
            done += step
        plsc.subcore_barrier()

        def group_main(g, carry):
            for b in range(RI):
                j = g * RI + b
                stage_a(j, b)
                stage_b(j, b)
                stage_c(j, b)
            return carry

        def group_tail(g, carry):
            for b in range(RI):
                j = g * RI + b
                pl.when(j + GD < ncw)(lambda: stage_a(j, b))
                pl.when(j < ncw)(lambda: stage_b(j, b))
                pl.when(j + RI - 1 < ncw)(lambda: stage_c(j, b))
            return carry

        lax.fori_loop(0, main_g, group_main, 0)
        lax.fori_loop(main_g, ngroups, group_tail, 0)
        pltpu.sync_copy(cnt, cnt_out.at[wid])
        plsc.subcore_barrier()
        pltpu.sync_copy(acc.at[pl.ds(s * zrows, zrows)],
                        sum_out.at[c, pl.ds(s * zrows, zrows)])

    return edge_pass


_edge_pass0 = _make_edge_pass(E0, S1_PAD, S1, 4, 8)
_edge_pass1 = _make_edge_pass(E1, S2_PAD, S2, 4, 8)


def _dense_body(last, p_ref, c_ref, xt_ref, wl_ref, bl_ref, wr_ref, o_ref):
    sums = p_ref[0] + p_ref[1]
    cnt = jnp.sum(c_ref[...], axis=0)[:, None]
    aggr = sums / jnp.maximum(cnt, 1.0)
    h = (jnp.dot(aggr, wl_ref[...], preferred_element_type=jnp.float32)
         + bl_ref[...]
         + jnp.dot(xt_ref[...], wr_ref[...], preferred_element_type=jnp.float32))
    if last:
        m = jnp.max(h, axis=-1, keepdims=True)
        o_ref[...] = (h - m) - jnp.log(
            jnp.sum(jnp.exp(h - m), axis=-1, keepdims=True))
    else:
        o_ref[...] = jnp.maximum(h, 0.0)


def _dense_layer(p, c, xt, wl, bl, wr, n_rows, last):
    blk = 1024
    grid = (n_rows + blk - 1) // blk
    return pl.pallas_call(
        functools.partial(_dense_body, last),
        grid=(grid,),
        in_specs=[
            pl.BlockSpec((NC, blk, D), lambda i: (0, i, 0)),
            pl.BlockSpec((NW, blk), lambda i: (0, i)),
            pl.BlockSpec((blk, D), lambda i: (i, 0)),
            pl.BlockSpec((D, D), lambda i: (0, 0)),
            pl.BlockSpec((1, D), lambda i: (0, 0)),
            pl.BlockSpec((D, D), lambda i: (0, 0)),
        ],
        out_specs=pl.BlockSpec((blk, D), lambda i: (i, 0)),
        out_shape=jax.ShapeDtypeStruct((n_rows, D), jnp.float32),
    )(p, c, xt, wl, bl, wr)


def kernel(x, row0, col0, row1, col1, size1, size2, Wl0, bl0, Wr0, Wl1, bl1, Wr1):
    p0, c0 = _edge_pass0(x, row0, col0)
    h = _dense_layer(p0, c0, x, Wl0, bl0.reshape(1, D), Wr0, S1, last=False)
    p1, c1 = _edge_pass1(h, row1, col1)
    out = _dense_layer(p1, c1, h, Wl1, bl1.reshape(1, D), Wr1, S2, last=True)
    return out
